# baseline (speedup 1.0000x reference)
import functools

import jax
import jax.numpy as jnp
from jax import lax
from jax.experimental import pallas as pl
from jax.experimental.pallas import tpu as pltpu
from jax.experimental.pallas import tpu_sc as plsc

N = 10000
NP = 10240
E = 320000
HALF = E // 2
EP = 327680
NC = 2
NS = 16
EPT = EP // (NC * NS)
CH = 128
NCHUNK = EPT // CH
RPT = NP // NS
DF = 128
DH = 64
NCLS = 16

_mesh = plsc.VectorSubcoreMesh(core_axis_name="c", subcore_axis_name="s")


def _sc_deg(ei3, p_pad):
    @functools.partial(
        pl.kernel,
        out_type=(jax.ShapeDtypeStruct((NC, NP), jnp.float32),
                  jax.ShapeDtypeStruct((NC * NS, NCHUNK, CH), jnp.float32),
                  jax.ShapeDtypeStruct((NC * NS, NCHUNK, CH), jnp.int32),
                  jax.ShapeDtypeStruct((NC * NS, NCHUNK, CH), jnp.int32)),
        mesh=_mesh,
        scratch_types=[
            pltpu.VMEM((NCHUNK, CH), jnp.int32),
            pltpu.VMEM((NCHUNK, CH), jnp.int32),
            pltpu.VMEM((NCHUNK, CH), jnp.float32),
            pltpu.VMEM((EPT,), jnp.float32),
            pltpu.VMEM((RPT,), jnp.float32),
            pltpu.VMEM_SHARED((NP,), jnp.float32),
            pltpu.SemaphoreType.DMA,
        ],
        compiler_params=pltpu.CompilerParams(use_tc_tiling_on_sc=False),
    )
    def k(ei_hbm, p_hbm, out_hbm, ew_out_hbm, src_out_hbm, dst_out_hbm,
          sidxv, didx, ewv, pbuf, zbuf, deg_sh, sem):
        c = lax.axis_index("c")
        s = lax.axis_index("s")
        w = c * NS + s
        last = NC * NS - 1
        real_rows = (E - last * EPT) // CH
        zero16 = jnp.zeros((16,), jnp.float32)
        one16 = jnp.full((16,), 1.0, jnp.float32)
        lane16 = jnp.arange(16, dtype=jnp.int32)

        def zb(j, _):
            zbuf[pl.ds(j * 16, 16)] = zero16
            return 0

        lax.fori_loop(0, RPT // 16, zb, 0)
        pltpu.sync_copy(zbuf, deg_sh.at[pl.ds(s * RPT, RPT)])

        @pl.when(w != last)
        def _():
            pltpu.sync_copy(ei_hbm.at[0, pl.ds(w * NCHUNK, NCHUNK)], sidxv)
            pltpu.sync_copy(ei_hbm.at[1, pl.ds(w * NCHUNK, NCHUNK)], didx)

        @pl.when(w == last)
        def _():
            pltpu.sync_copy(ei_hbm.at[0, pl.ds(last * NCHUNK, real_rows)],
                            sidxv.at[pl.ds(0, real_rows)])
            pltpu.sync_copy(ei_hbm.at[1, pl.ds(last * NCHUNK, real_rows)],
                            didx.at[pl.ds(0, real_rows)])

            def padrow(i, _):
                for jj in range(CH // 16):
                    v = N + (i * CH + jj * 16 + lane16) % (NP - N)
                    sidxv[i, pl.ds(jj * 16, 16)] = v
                    didx[i, pl.ds(jj * 16, 16)] = v
                return 0

            lax.fori_loop(real_rows, NCHUNK, padrow, 0)

        pltpu.sync_copy(sidxv, src_out_hbm.at[w])
        pltpu.sync_copy(didx, dst_out_hbm.at[w])

        straddle = NS - 1
        cut = HALF - straddle * EPT

        @pl.when(w != straddle)
        def _():
            pb = pl.multiple_of(jnp.where(w * EPT >= HALF,
                                          w * EPT - HALF, w * EPT), CH)
            pltpu.sync_copy(p_hbm.at[pl.ds(pb, EPT)], pbuf)

        @pl.when(w == straddle)
        def _():
            pltpu.sync_copy(p_hbm.at[pl.ds(straddle * EPT, cut)],
                            pbuf.at[pl.ds(0, cut)])
            pltpu.sync_copy(p_hbm.at[pl.ds(0, EPT - cut)],
                            pbuf.at[pl.ds(cut, EPT - cut)])

        def sig_row(i, _):
            for jj in range(CH // 16):
                v = pbuf[pl.ds(i * CH + jj * 16, 16)]
                ewv[i, pl.ds(jj * 16, 16)] = one16 / (one16 + jnp.exp(-v))
            return 0

        lax.fori_loop(0, NCHUNK, sig_row, 0)
        pltpu.sync_copy(ewv, ew_out_hbm.at[w])
        plsc.subcore_barrier()

        GRP = 8

        def grp_body(p, _):
            descs = []
            for q in range(GRP):
                i = p * GRP + q
                d = pltpu.make_async_copy(ewv.at[i], deg_sh.at[didx.at[i]], sem)
                d.start(add=True)
                descs.append(d)
            for d in descs:
                d.wait()
            return 0

        lax.fori_loop(0, NCHUNK // GRP, grp_body, 0)
        plsc.subcore_barrier()
        pltpu.sync_copy(deg_sh.at[pl.ds(s * RPT, RPT)],
                        out_hbm.at[c, pl.ds(s * RPT, RPT)])

    return k(ei3, p_pad)


def _sc_msg(g, src3, dst3, ew3, d):
    grp = d // 16

    @functools.partial(
        pl.kernel,
        out_type=jax.ShapeDtypeStruct((NC, NP, d), jnp.float32),
        mesh=_mesh,
        scratch_types=[
            pltpu.VMEM((NCHUNK, CH), jnp.int32),
            pltpu.VMEM((NCHUNK, CH), jnp.int32),
            pltpu.VMEM((NCHUNK, CH), jnp.float32),
            pltpu.VMEM((CH, d), jnp.float32),
            pltpu.VMEM((CH, d), jnp.float32),
            pltpu.VMEM((CH, d), jnp.float32),
            pltpu.VMEM((CH, d), jnp.float32),
            pltpu.VMEM_SHARED((NP, d), jnp.float32),
            pltpu.SemaphoreType.DMA,
            pltpu.SemaphoreType.DMA,
            pltpu.SemaphoreType.DMA,
            pltpu.SemaphoreType.DMA,
        ],
        compiler_params=(None if d % 128 == 0
                         else pltpu.CompilerParams(use_tc_tiling_on_sc=False)),
    )
    def k(g_hbm, src_hbm, dst_hbm, ew_hbm, out_hbm,
          sidx, didx, ewv, rg0, rg1, rs0, rs1, acc_sh, gs0, gs1, ss0, ss1):
        c = lax.axis_index("c")
        s = lax.axis_index("s")
        w = c * NS + s
        zero16 = jnp.zeros((16,), jnp.float32)
        rg = (rg0, rg1)
        rs = (rs0, rs1)
        gsem = (gs0, gs1)
        ssem = (ss0, ss1)

        def gdesc(i, b):
            return pltpu.make_async_copy(g_hbm.at[sidx.at[i]], rg[b], gsem[b])

        def sdesc(i, b):
            return pltpu.make_async_copy(rs[b], acc_sh.at[didx.at[i]], ssem[b])

        def scale(ii, b):
            rgb, rsb = rg[b], rs[b]

            def grp_body(jj, _):
                v = ewv[ii, pl.ds(jj * 16, 16)]
                for l in range(16):
                    wt = v[l]
                    j = jj * 16 + l
                    for kk in range(grp):
                        sl = pl.ds(kk * 16, 16)
                        rsb[j, sl] = rgb[j, sl] * wt
                return 0

            lax.fori_loop(0, CH // 16, grp_body, 0)

        pltpu.sync_copy(src_hbm.at[w], sidx)
        pltpu.sync_copy(dst_hbm.at[w], didx)
        pltpu.sync_copy(ew_hbm.at[w], ewv)

        def zr(j, _):
            for kk in range(grp):
                rs0[j, pl.ds(kk * 16, 16)] = zero16
            return 0

        lax.fori_loop(0, CH, zr, 0)
        for bb in range(RPT // CH):
            pltpu.sync_copy(rs0, acc_sh.at[pl.ds(s * RPT + bb * CH, CH)])
        plsc.subcore_barrier()

        gdesc(0, 0).start()
        gdesc(1, 1).start()
        for b in range(2):
            gdesc(b, b).wait()
            scale(b, b)
            sdesc(b, b).start(add=True)
            gdesc(b + 2, b).start()

        def body(p, _):
            for b in range(2):
                ii = 2 * p + b
                sdesc(ii, b).wait()
                gdesc(ii, b).wait()
                scale(ii, b)
                sdesc(ii, b).start(add=True)
                gdesc(ii + 2, b).start()
            return 0

        lax.fori_loop(1, NCHUNK // 2 - 1, body, 0)

        for b in range(2):
            ii = NCHUNK - 2 + b
            sdesc(ii, b).wait()
            gdesc(ii, b).wait()
            scale(ii, b)
            sdesc(ii, b).start(add=True)
        for b in range(2):
            sdesc(NCHUNK - 2 + b, b).wait()

        plsc.subcore_barrier()
        pltpu.sync_copy(acc_sh.at[pl.ds(s * RPT, RPT)],
                        out_hbm.at[c, pl.ds(s * RPT, RPT)])

    return k(g, src3, dst3, ew3)


_BLK = 2048


def _tc_first(deg_t, x_pad, W1):
    def k(dp_ref, x_ref, w_ref, glo_ref, ghi_ref, dinv_ref):
        deg = dp_ref[:, 0:1] + dp_ref[:, 1:2] + 1.0
        dinv = lax.rsqrt(deg)
        dinv_ref[...] = dinv
        g = dinv * jnp.dot(x_ref[...], w_ref[...],
                           preferred_element_type=jnp.float32)
        glo_ref[...] = g[:, :DH]
        ghi_ref[...] = g[:, DH:]

    return pl.pallas_call(
        k,
        grid=(NP // _BLK,),
        in_specs=[
            pl.BlockSpec((_BLK, NC), lambda i: (i, 0)),
            pl.BlockSpec((_BLK, DF), lambda i: (i, 0)),
            pl.BlockSpec((DF, DF), lambda i: (0, 0)),
        ],
        out_specs=[
            pl.BlockSpec((_BLK, DH), lambda i: (i, 0)),
            pl.BlockSpec((_BLK, DH), lambda i: (i, 0)),
            pl.BlockSpec((_BLK, 1), lambda i: (i, 0)),
        ],
        out_shape=[
            jax.ShapeDtypeStruct((NP, DH), jnp.float32),
            jax.ShapeDtypeStruct((NP, DH), jnp.float32),
            jax.ShapeDtypeStruct((NP, 1), jnp.float32),
        ],
    )(deg_t, x_pad, W1)


def _tc_mid(acc_lo, acc_hi, g_lo, g_hi, dinv, W, dout):
    split = dout == DF

    def k(al_ref, ah_ref, gl_ref, gh_ref, d_ref, w_ref, *o_refs):
        dinv = d_ref[...]
        hlo = jnp.maximum(dinv * (al_ref[0] + al_ref[1] + gl_ref[...]), 0.0)
        hhi = jnp.maximum(dinv * (ah_ref[0] + ah_ref[1] + gh_ref[...]), 0.0)
        h = jnp.concatenate([hlo, hhi], axis=-1)
        r = dinv * jnp.dot(h, w_ref[...], preferred_element_type=jnp.float32)
        if split:
            o_refs[0][...] = r[:, :DH]
            o_refs[1][...] = r[:, DH:]
        else:
            o_refs[0][...] = r

    if split:
        out_specs = [pl.BlockSpec((_BLK, DH), lambda i: (i, 0)),
                     pl.BlockSpec((_BLK, DH), lambda i: (i, 0))]
        out_shape = [jax.ShapeDtypeStruct((NP, DH), jnp.float32),
                     jax.ShapeDtypeStruct((NP, DH), jnp.float32)]
    else:
        out_specs = pl.BlockSpec((_BLK, dout), lambda i: (i, 0))
        out_shape = jax.ShapeDtypeStruct((NP, dout), jnp.float32)

    return pl.pallas_call(
        k,
        grid=(NP // _BLK,),
        in_specs=[
            pl.BlockSpec((NC, _BLK, DH), lambda i: (0, i, 0)),
            pl.BlockSpec((NC, _BLK, DH), lambda i: (0, i, 0)),
            pl.BlockSpec((_BLK, DH), lambda i: (i, 0)),
            pl.BlockSpec((_BLK, DH), lambda i: (i, 0)),
            pl.BlockSpec((_BLK, 1), lambda i: (i, 0)),
            pl.BlockSpec((DF, dout), lambda i: (0, 0)),
        ],
        out_specs=out_specs,
        out_shape=out_shape,
    )(acc_lo, acc_hi, g_lo, g_hi, dinv, W)


def _tc_last(acc3, g3, dinv, idx_arr):
    def k(i_ref, a_ref, g_ref, d_ref, o_ref):
        idx = i_ref[0]
        row = d_ref[pl.ds(idx, 1), :] * (a_ref[0, pl.ds(idx, 1), :]
                                         + a_ref[1, pl.ds(idx, 1), :]
                                         + g_ref[pl.ds(idx, 1), :])
        m = jnp.max(row)
        lse = m + jnp.log(jnp.sum(jnp.exp(row - m)))
        o_ref[...] = row - lse

    return pl.pallas_call(
        k,
        in_specs=[
            pl.BlockSpec(memory_space=pltpu.SMEM),
            pl.BlockSpec(memory_space=pltpu.VMEM),
            pl.BlockSpec(memory_space=pltpu.VMEM),
            pl.BlockSpec(memory_space=pltpu.VMEM),
        ],
        out_shape=jax.ShapeDtypeStruct((1, NCLS), jnp.float32),
    )(idx_arr, acc3, g3, dinv)


def kernel(x, edge_index, index, P_vec, W1, W2, W3):
    ei3 = edge_index.astype(jnp.int32).reshape(2, E // CH, CH)

    p_pad = jnp.pad(P_vec.astype(jnp.float32), (0, EP - E))

    x_pad = jnp.pad(x.astype(jnp.float32), ((0, NP - N), (0, 0)))

    deg_parts, ew3, src3, dst3 = _sc_deg(ei3, p_pad)
    g1lo, g1hi, dinv = _tc_first(deg_parts.T, x_pad, W1)
    acc1a = _sc_msg(g1lo, src3, dst3, ew3, DH)
    acc1b = _sc_msg(g1hi, src3, dst3, ew3, DH)
    g2lo, g2hi = _tc_mid(acc1a, acc1b, g1lo, g1hi, dinv, W2, DF)
    acc2a = _sc_msg(g2lo, src3, dst3, ew3, DH)
    acc2b = _sc_msg(g2hi, src3, dst3, ew3, DH)
    g3 = _tc_mid(acc2a, acc2b, g2lo, g2hi, dinv, W3, NCLS)
    acc3 = _sc_msg(g3, src3, dst3, ew3, NCLS)

    idx_arr = jnp.reshape(jnp.asarray(index, jnp.int32), (1,))
    out = _tc_last(acc3, g3, dinv, idx_arr)
    return out.reshape(NCLS)


# speedup vs baseline: 1.0099x; 1.0099x over previous
import functools

import jax
import jax.numpy as jnp
from jax import lax
from jax.experimental import pallas as pl
from jax.experimental.pallas import tpu as pltpu
from jax.experimental.pallas import tpu_sc as plsc

N = 10000
NP = 10240
E = 320000
HALF = E // 2
EP = 327680
NC = 2
NS = 16
EPT = EP // (NC * NS)
CH = 128
NCHUNK = EPT // CH
RPT = NP // NS
DF = 128
DH = 64
NCLS = 16

_mesh = plsc.VectorSubcoreMesh(core_axis_name="c", subcore_axis_name="s")


def _sc_deg(ei3, p_pad):
    @functools.partial(
        pl.kernel,
        out_type=(jax.ShapeDtypeStruct((NC, NP), jnp.float32),
                  jax.ShapeDtypeStruct((NC * NS, NCHUNK, CH), jnp.float32),
                  jax.ShapeDtypeStruct((NC * NS, NCHUNK, CH), jnp.int32),
                  jax.ShapeDtypeStruct((NC * NS, NCHUNK, CH), jnp.int32)),
        mesh=_mesh,
        scratch_types=[
            pltpu.VMEM((NCHUNK, CH), jnp.int32),
            pltpu.VMEM((NCHUNK, CH), jnp.int32),
            pltpu.VMEM((NCHUNK, CH), jnp.float32),
            pltpu.VMEM((EPT,), jnp.float32),
            pltpu.VMEM((RPT,), jnp.float32),
            pltpu.VMEM_SHARED((NP,), jnp.float32),
            pltpu.SemaphoreType.DMA,
        ],
        compiler_params=pltpu.CompilerParams(use_tc_tiling_on_sc=False),
    )
    def k(ei_hbm, p_hbm, out_hbm, ew_out_hbm, src_out_hbm, dst_out_hbm,
          sidxv, didx, ewv, pbuf, zbuf, deg_sh, sem):
        c = lax.axis_index("c")
        s = lax.axis_index("s")
        w = c * NS + s
        last = NC * NS - 1
        real_rows = (E - last * EPT) // CH
        zero16 = jnp.zeros((16,), jnp.float32)
        one16 = jnp.full((16,), 1.0, jnp.float32)
        lane16 = jnp.arange(16, dtype=jnp.int32)

        def zb(j, _):
            zbuf[pl.ds(j * 16, 16)] = zero16
            return 0

        lax.fori_loop(0, RPT // 16, zb, 0)
        pltpu.sync_copy(zbuf, deg_sh.at[pl.ds(s * RPT, RPT)])

        @pl.when(w != last)
        def _():
            pltpu.sync_copy(ei_hbm.at[0, pl.ds(w * NCHUNK, NCHUNK)], sidxv)
            pltpu.sync_copy(ei_hbm.at[1, pl.ds(w * NCHUNK, NCHUNK)], didx)

        @pl.when(w == last)
        def _():
            pltpu.sync_copy(ei_hbm.at[0, pl.ds(last * NCHUNK, real_rows)],
                            sidxv.at[pl.ds(0, real_rows)])
            pltpu.sync_copy(ei_hbm.at[1, pl.ds(last * NCHUNK, real_rows)],
                            didx.at[pl.ds(0, real_rows)])

            def padrow(i, _):
                for jj in range(CH // 16):
                    v = N + (i * CH + jj * 16 + lane16) % (NP - N)
                    sidxv[i, pl.ds(jj * 16, 16)] = v
                    didx[i, pl.ds(jj * 16, 16)] = v
                return 0

            lax.fori_loop(real_rows, NCHUNK, padrow, 0)

        pltpu.sync_copy(sidxv, src_out_hbm.at[w])
        pltpu.sync_copy(didx, dst_out_hbm.at[w])

        straddle = NS - 1
        cut = HALF - straddle * EPT

        @pl.when(w != straddle)
        def _():
            pb = pl.multiple_of(jnp.where(w * EPT >= HALF,
                                          w * EPT - HALF, w * EPT), CH)
            pltpu.sync_copy(p_hbm.at[pl.ds(pb, EPT)], pbuf)

        @pl.when(w == straddle)
        def _():
            pltpu.sync_copy(p_hbm.at[pl.ds(straddle * EPT, cut)],
                            pbuf.at[pl.ds(0, cut)])
            pltpu.sync_copy(p_hbm.at[pl.ds(0, EPT - cut)],
                            pbuf.at[pl.ds(cut, EPT - cut)])

        def sig_row(i, _):
            for jj in range(CH // 16):
                v = pbuf[pl.ds(i * CH + jj * 16, 16)]
                ewv[i, pl.ds(jj * 16, 16)] = one16 / (one16 + jnp.exp(-v))
            return 0

        lax.fori_loop(0, NCHUNK, sig_row, 0)
        pltpu.sync_copy(ewv, ew_out_hbm.at[w])
        plsc.subcore_barrier()

        GRP = 8

        def grp_body(p, _):
            descs = []
            for q in range(GRP):
                i = p * GRP + q
                d = pltpu.make_async_copy(ewv.at[i], deg_sh.at[didx.at[i]], sem)
                d.start(add=True)
                descs.append(d)
            for d in descs:
                d.wait()
            return 0

        lax.fori_loop(0, NCHUNK // GRP, grp_body, 0)
        plsc.subcore_barrier()
        pltpu.sync_copy(deg_sh.at[pl.ds(s * RPT, RPT)],
                        out_hbm.at[c, pl.ds(s * RPT, RPT)])

    return k(ei3, p_pad)


def _sc_msg(g, src3, dst3, ew3, d):
    grp = d // 16
    SB = 2
    NSUP = NCHUNK // SB
    HROWS = NCHUNK // 2
    HSUP = NSUP // 2

    @functools.partial(
        pl.kernel,
        out_type=jax.ShapeDtypeStruct((NC, NP, d), jnp.float32),
        mesh=_mesh,
        scratch_types=[
            pltpu.VMEM((HROWS, CH), jnp.int32),
            pltpu.VMEM((HROWS, CH), jnp.int32),
            pltpu.VMEM((HROWS, CH), jnp.float32),
            pltpu.VMEM((SB * CH, d), jnp.float32),
            pltpu.VMEM((SB * CH, d), jnp.float32),
            pltpu.VMEM((SB * CH, d), jnp.float32),
            pltpu.VMEM((SB * CH, d), jnp.float32),
            pltpu.VMEM_SHARED((NP, d), jnp.float32),
            pltpu.SemaphoreType.DMA,
            pltpu.SemaphoreType.DMA,
            pltpu.SemaphoreType.DMA,
            pltpu.SemaphoreType.DMA,
        ],
        compiler_params=pltpu.CompilerParams(use_tc_tiling_on_sc=False),
    )
    def k(g_hbm, src_hbm, dst_hbm, ew_hbm, out_hbm,
          sidx, didx, ewv, rg0, rg1, rs0, rs1, acc_sh, gs0, gs1, ss0, ss1):
        c = lax.axis_index("c")
        s = lax.axis_index("s")
        w = c * NS + s
        zero16 = jnp.zeros((16,), jnp.float32)
        rg = (rg0, rg1)
        rs = (rs0, rs1)
        gsem = (gs0, gs1)
        ssem = (ss0, ss1)

        def gstart(lr, b):
            for q in range(SB):
                pltpu.make_async_copy(g_hbm.at[sidx.at[lr + q]],
                                      rg[b].at[pl.ds(q * CH, CH)],
                                      gsem[b]).start()

        def gwait(b):
            for q in range(SB):
                pltpu.make_async_copy(g_hbm.at[sidx.at[q]],
                                      rg[b].at[pl.ds(q * CH, CH)],
                                      gsem[b]).wait()

        def sstart(lr, b):
            for q in range(SB):
                pltpu.make_async_copy(rs[b].at[pl.ds(q * CH, CH)],
                                      acc_sh.at[didx.at[lr + q]],
                                      ssem[b]).start(add=True)

        def swait(b):
            for q in range(SB):
                pltpu.make_async_copy(rs[b].at[pl.ds(q * CH, CH)],
                                      acc_sh.at[didx.at[q]],
                                      ssem[b]).wait()

        def scale(lr, b):
            rgb, rsb = rg[b], rs[b]

            def grp_body(jj, _):
                for q in range(SB):
                    v = ewv[lr + q, pl.ds(jj * 16, 16)]
                    for l in range(16):
                        wt = v[l]
                        j = q * CH + jj * 16 + l
                        for kk in range(grp):
                            sl = pl.ds(kk * 16, 16)
                            rsb[j, sl] = rgb[j, sl] * wt
                return 0

            lax.fori_loop(0, CH // 16, grp_body, 0)

        def refill(half):
            pltpu.sync_copy(src_hbm.at[w, pl.ds(half * HROWS, HROWS)], sidx)
            pltpu.sync_copy(dst_hbm.at[w, pl.ds(half * HROWS, HROWS)], didx)
            pltpu.sync_copy(ew_hbm.at[w, pl.ds(half * HROWS, HROWS)], ewv)

        def head(base):
            gstart(0, 0)
            gstart(SB, 1)
            for b in range(2):
                gwait(b)
                scale(SB * b, b)
                sstart(SB * b, b)
                gstart(SB * (b + 2), b)

        def steady_body(p, off):
            for b in range(2):
                sp = 2 * p + b
                lr = SB * (sp - off)
                swait(b)
                gwait(b)
                scale(lr, b)
                sstart(lr, b)
                gstart(lr + 2 * SB, b)
            return off

        def boundary_and_tail(last_two_lr):
            for b in range(2):
                lr = last_two_lr + SB * b
                swait(b)
                gwait(b)
                scale(lr, b)
                sstart(lr, b)
            for b in range(2):
                swait(b)

        def zr(j, _):
            for kk in range(grp):
                rs0[j, pl.ds(kk * 16, 16)] = zero16
            return 0

        lax.fori_loop(0, CH, zr, 0)
        for bb in range(RPT // CH):
            pltpu.sync_copy(rs0.at[pl.ds(0, CH)],
                            acc_sh.at[pl.ds(s * RPT + bb * CH, CH)])
        refill(0)
        plsc.subcore_barrier()

        head(0)
        lax.fori_loop(1, HSUP // 2 - 1, lambda p, o: steady_body(p, o), 0)
        boundary_and_tail(SB * (HSUP - 2))

        refill(1)
        head(HSUP)
        lax.fori_loop(HSUP // 2 + 1, HSUP - 1,
                      lambda p, o: steady_body(p, o), HSUP)
        boundary_and_tail(SB * (HSUP - 2))

        plsc.subcore_barrier()
        pltpu.sync_copy(acc_sh.at[pl.ds(s * RPT, RPT)],
                        out_hbm.at[c, pl.ds(s * RPT, RPT)])

    return k(g, src3, dst3, ew3)


_BLK = 2048


def _tc_first(deg_t, x_pad, W1):
    def k(dp_ref, x_ref, w_ref, glo_ref, ghi_ref, dinv_ref):
        deg = dp_ref[:, 0:1] + dp_ref[:, 1:2] + 1.0
        dinv = lax.rsqrt(deg)
        dinv_ref[...] = dinv
        g = dinv * jnp.dot(x_ref[...], w_ref[...],
                           preferred_element_type=jnp.float32)
        glo_ref[...] = g[:, :DH]
        ghi_ref[...] = g[:, DH:]

    return pl.pallas_call(
        k,
        grid=(NP // _BLK,),
        in_specs=[
            pl.BlockSpec((_BLK, NC), lambda i: (i, 0)),
            pl.BlockSpec((_BLK, DF), lambda i: (i, 0)),
            pl.BlockSpec((DF, DF), lambda i: (0, 0)),
        ],
        out_specs=[
            pl.BlockSpec((_BLK, DH), lambda i: (i, 0)),
            pl.BlockSpec((_BLK, DH), lambda i: (i, 0)),
            pl.BlockSpec((_BLK, 1), lambda i: (i, 0)),
        ],
        out_shape=[
            jax.ShapeDtypeStruct((NP, DH), jnp.float32),
            jax.ShapeDtypeStruct((NP, DH), jnp.float32),
            jax.ShapeDtypeStruct((NP, 1), jnp.float32),
        ],
    )(deg_t, x_pad, W1)


def _tc_mid(acc_lo, acc_hi, g_lo, g_hi, dinv, W, dout):
    split = dout == DF

    def k(al_ref, ah_ref, gl_ref, gh_ref, d_ref, w_ref, *o_refs):
        dinv = d_ref[...]
        hlo = jnp.maximum(dinv * (al_ref[0] + al_ref[1] + gl_ref[...]), 0.0)
        hhi = jnp.maximum(dinv * (ah_ref[0] + ah_ref[1] + gh_ref[...]), 0.0)
        h = jnp.concatenate([hlo, hhi], axis=-1)
        r = dinv * jnp.dot(h, w_ref[...], preferred_element_type=jnp.float32)
        if split:
            o_refs[0][...] = r[:, :DH]
            o_refs[1][...] = r[:, DH:]
        else:
            o_refs[0][...] = r

    if split:
        out_specs = [pl.BlockSpec((_BLK, DH), lambda i: (i, 0)),
                     pl.BlockSpec((_BLK, DH), lambda i: (i, 0))]
        out_shape = [jax.ShapeDtypeStruct((NP, DH), jnp.float32),
                     jax.ShapeDtypeStruct((NP, DH), jnp.float32)]
    else:
        out_specs = pl.BlockSpec((_BLK, dout), lambda i: (i, 0))
        out_shape = jax.ShapeDtypeStruct((NP, dout), jnp.float32)

    return pl.pallas_call(
        k,
        grid=(NP // _BLK,),
        in_specs=[
            pl.BlockSpec((NC, _BLK, DH), lambda i: (0, i, 0)),
            pl.BlockSpec((NC, _BLK, DH), lambda i: (0, i, 0)),
            pl.BlockSpec((_BLK, DH), lambda i: (i, 0)),
            pl.BlockSpec((_BLK, DH), lambda i: (i, 0)),
            pl.BlockSpec((_BLK, 1), lambda i: (i, 0)),
            pl.BlockSpec((DF, dout), lambda i: (0, 0)),
        ],
        out_specs=out_specs,
        out_shape=out_shape,
    )(acc_lo, acc_hi, g_lo, g_hi, dinv, W)


def _tc_last(acc3, g3, dinv, idx_arr):
    def k(i_ref, a_ref, g_ref, d_ref, o_ref):
        idx = i_ref[0]
        row = d_ref[pl.ds(idx, 1), :] * (a_ref[0, pl.ds(idx, 1), :]
                                         + a_ref[1, pl.ds(idx, 1), :]
                                         + g_ref[pl.ds(idx, 1), :])
        m = jnp.max(row)
        lse = m + jnp.log(jnp.sum(jnp.exp(row - m)))
        o_ref[...] = row - lse

    return pl.pallas_call(
        k,
        in_specs=[
            pl.BlockSpec(memory_space=pltpu.SMEM),
            pl.BlockSpec(memory_space=pltpu.VMEM),
            pl.BlockSpec(memory_space=pltpu.VMEM),
            pl.BlockSpec(memory_space=pltpu.VMEM),
        ],
        out_shape=jax.ShapeDtypeStruct((1, NCLS), jnp.float32),
    )(idx_arr, acc3, g3, dinv)


def kernel(x, edge_index, index, P_vec, W1, W2, W3):
    ei3 = edge_index.astype(jnp.int32).reshape(2, E // CH, CH)

    p_pad = jnp.pad(P_vec.astype(jnp.float32), (0, EP - E))

    x_pad = jnp.pad(x.astype(jnp.float32), ((0, NP - N), (0, 0)))

    deg_parts, ew3, src3, dst3 = _sc_deg(ei3, p_pad)
    g1lo, g1hi, dinv = _tc_first(deg_parts.T, x_pad, W1)
    acc1a = _sc_msg(g1lo, src3, dst3, ew3, DH)
    acc1b = _sc_msg(g1hi, src3, dst3, ew3, DH)
    g2lo, g2hi = _tc_mid(acc1a, acc1b, g1lo, g1hi, dinv, W2, DF)
    acc2a = _sc_msg(g2lo, src3, dst3, ew3, DH)
    acc2b = _sc_msg(g2hi, src3, dst3, ew3, DH)
    g3 = _tc_mid(acc2a, acc2b, g2lo, g2hi, dinv, W3, NCLS)
    acc3 = _sc_msg(g3, src3, dst3, ew3, NCLS)

    idx_arr = jnp.reshape(jnp.asarray(index, jnp.int32), (1,))
    out = _tc_last(acc3, g3, dinv, idx_arr)
    return out.reshape(NCLS)
